# baseline (speedup 1.0000x reference)
import functools

import jax
import jax.numpy as jnp
from jax import lax
from jax.experimental import pallas as pl
from jax.experimental.pallas import tpu as pltpu
from jax.experimental.pallas import tpu_sc as plsc

B = 4096
F = 26
D = 32
NW = 32
BPW = B // NW
ROWS = BPW * F
OUTW = D + 1
LANES = 16
HALF = 16
VR = (1000000 * D) // HALF
NF = 3
GROUPS = [list(range(s, min(s + NF, F))) for s in range(0, F, NF)]


def _build():
    mesh = plsc.VectorSubcoreMesh(core_axis_name="c", subcore_axis_name="s")

    @functools.partial(
        pl.kernel,
        out_type=jax.ShapeDtypeStruct((B * OUTW,), jnp.float32),
        mesh=mesh,
        compiler_params=pltpu.CompilerParams(use_tc_tiling_on_sc=False),
        scratch_types=[
            pltpu.VMEM((F, BPW), jnp.int32),
            pltpu.VMEM((F, BPW), jnp.int32),
            pltpu.VMEM((F, BPW), jnp.int32),
            *[pltpu.VMEM((BPW, HALF), jnp.float32) for _ in range(4 * NF)],
            pltpu.VMEM((BPW * OUTW,), jnp.float32),
            pltpu.SemaphoreType.DMA,
            pltpu.SemaphoreType.DMA,
        ],
    )
    def sc_kernel(idx_hbm, table_hbm, out_hbm, idx_v, ev_v, od_v, *rest):
        bufs, (out_v, s0, s1) = rest[: 4 * NF], rest[4 * NF:]

        def buf_ev(par, fl):
            return bufs[(par * NF + fl) * 2]

        def buf_od(par, fl):
            return bufs[(par * NF + fl) * 2 + 1]

        wid = lax.axis_index("s") * 2 + lax.axis_index("c")
        sems = (s0, s1)

        pltpu.sync_copy(idx_hbm.at[:, pl.ds(wid * BPW, BPW)], idx_v)

        def shift_body(f, carry):
            for j in range(BPW // LANES):
                v = idx_v[f, pl.ds(j * LANES, LANES)]
                e = lax.shift_left(v, 1)
                ev_v[f, pl.ds(j * LANES, LANES)] = e
                od_v[f, pl.ds(j * LANES, LANES)] = e + 1
            return carry

        lax.fori_loop(0, F, shift_body, 0)

        def fire(g):
            par = g % 2
            for fl, f in enumerate(GROUPS[g]):
                pltpu.async_copy(
                    table_hbm.at[ev_v.at[f]],
                    buf_ev(par, fl),
                    sems[par],
                )
                pltpu.async_copy(
                    table_hbm.at[od_v.at[f]],
                    buf_od(par, fl),
                    sems[par],
                )

        def drain(g):
            par = g % 2
            for fl, f in enumerate(GROUPS[g]):
                pltpu.make_async_copy(
                    table_hbm.at[ev_v.at[f]],
                    buf_ev(par, fl),
                    sems[par],
                ).wait()
                pltpu.make_async_copy(
                    table_hbm.at[od_v.at[f]],
                    buf_od(par, fl),
                    sems[par],
                ).wait()

        def pool(g):
            par = g % 2
            first = g == 0
            nf = len(GROUPS[g])

            def body(b, carry):
                o = b * OUTW
                if first:
                    out_v[pl.ds(o, LANES)] = jnp.ones((LANES,), jnp.float32)
                    acc_lo = buf_ev(par, 0)[b, pl.ds(0, LANES)]
                    acc_hi = buf_od(par, 0)[b, pl.ds(0, LANES)]
                    start = 1
                else:
                    acc_lo = out_v[pl.ds(o + 1, LANES)]
                    acc_hi = out_v[pl.ds(o + 1 + LANES, LANES)]
                    start = 0
                for fl in range(start, nf):
                    acc_lo = acc_lo + buf_ev(par, fl)[b, pl.ds(0, LANES)]
                    acc_hi = acc_hi + buf_od(par, fl)[b, pl.ds(0, LANES)]
                out_v[pl.ds(o + 1, LANES)] = acc_lo
                out_v[pl.ds(o + 1 + LANES, LANES)] = acc_hi
                return carry

            lax.fori_loop(0, BPW, body, 0)

        fire(0)
        for g in range(1, len(GROUPS)):
            fire(g)
            drain(g - 1)
            pool(g - 1)
        last = len(GROUPS) - 1
        drain(last)
        pool(last)

        pltpu.sync_copy(
            out_v, out_hbm.at[pl.ds(wid * (BPW * OUTW), BPW * OUTW)]
        )

    return sc_kernel


_SC_KERNEL = _build()

TCB = 32768
WIDE = 128


def _tc_linearize_body(x_ref, o_ref):
    t = jax.lax.dot_general(
        x_ref[...], jnp.eye(D, dtype=jnp.float32),
        (((0,), (0,)), ((), ())),
        preferred_element_type=jnp.float32,
    )
    t = t.reshape(TCB // 4, 4, D)
    for q in range(4):
        o_ref[:, q * D:(q + 1) * D] = t[:, q, :]


_TC_LINEARIZE = pl.pallas_call(
    _tc_linearize_body,
    grid=((1000000 + TCB - 1) // TCB,),
    in_specs=[pl.BlockSpec((D, TCB), lambda i: (0, i))],
    out_specs=pl.BlockSpec((TCB // 4, WIDE), lambda i: (i, 0)),
    out_shape=jax.ShapeDtypeStruct((1000000 // 4, WIDE), jnp.float32),
)


@jax.jit
def kernel(user_feature_ids, embedding_weight):
    idx_t = jnp.asarray(user_feature_ids, jnp.int32).T
    table_lin = _TC_LINEARIZE(embedding_weight.T)
    table2 = table_lin.reshape(VR, HALF)
    flat = _SC_KERNEL(idx_t, table2)
    return flat.reshape(B, OUTW)


# speedup vs baseline: 1.3428x; 1.3428x over previous
import functools

import jax
import jax.numpy as jnp
from jax import lax
from jax.experimental import pallas as pl
from jax.experimental.pallas import tpu as pltpu
from jax.experimental.pallas import tpu_sc as plsc

B = 4096
F = 26
D = 32
NW = 32
BPW = B // NW
ROWS = BPW * F
OUTW = D + 1
LANES = 16
HALF = 16
QS = 245760
QROWS = 270336
VR = QROWS * 8
NF = 3
GROUPS = [list(range(s, min(s + NF, F))) for s in range(0, F, NF)]


def _build():
    mesh = plsc.VectorSubcoreMesh(core_axis_name="c", subcore_axis_name="s")

    @functools.partial(
        pl.kernel,
        out_type=jax.ShapeDtypeStruct((B * OUTW,), jnp.float32),
        mesh=mesh,
        compiler_params=pltpu.CompilerParams(use_tc_tiling_on_sc=False),
        scratch_types=[
            pltpu.VMEM((F, BPW), jnp.int32),
            pltpu.VMEM((F, BPW), jnp.int32),
            pltpu.VMEM((F, BPW), jnp.int32),
            *[pltpu.VMEM((BPW, HALF), jnp.float32) for _ in range(4 * NF)],
            pltpu.VMEM((BPW * OUTW,), jnp.float32),
            pltpu.SemaphoreType.DMA,
            pltpu.SemaphoreType.DMA,
        ],
    )
    def sc_kernel(idx_hbm, table_hbm, out_hbm, idx_v, ev_v, od_v, *rest):
        bufs, (out_v, s0, s1) = rest[: 4 * NF], rest[4 * NF:]

        def buf_ev(par, fl):
            return bufs[(par * NF + fl) * 2]

        def buf_od(par, fl):
            return bufs[(par * NF + fl) * 2 + 1]

        wid = lax.axis_index("s") * 2 + lax.axis_index("c")
        sems = (s0, s1)

        pltpu.sync_copy(idx_hbm.at[:, pl.ds(wid * BPW, BPW)], idx_v)

        def shift_body(f, carry):
            for j in range(BPW // LANES):
                v = idx_v[f, pl.ds(j * LANES, LANES)]
                one = jnp.ones((LANES,), jnp.int32)
                zero = jnp.zeros((LANES,), jnp.int32)
                q = (
                    jnp.where(v >= QS, one, zero)
                    + jnp.where(v >= 2 * QS, one, zero)
                    + jnp.where(v >= 3 * QS, one, zero)
                )
                e = lax.shift_left(v - q * QS, 3) + lax.shift_left(q, 1)
                ev_v[f, pl.ds(j * LANES, LANES)] = e
                od_v[f, pl.ds(j * LANES, LANES)] = e + 1
            return carry

        lax.fori_loop(0, F, shift_body, 0)

        def fire(g):
            par = g % 2
            for fl, f in enumerate(GROUPS[g]):
                pltpu.async_copy(
                    table_hbm.at[ev_v.at[f]],
                    buf_ev(par, fl),
                    sems[par],
                )
                pltpu.async_copy(
                    table_hbm.at[od_v.at[f]],
                    buf_od(par, fl),
                    sems[par],
                )

        def drain(g):
            par = g % 2
            for fl, f in enumerate(GROUPS[g]):
                pltpu.make_async_copy(
                    table_hbm.at[ev_v.at[f]],
                    buf_ev(par, fl),
                    sems[par],
                ).wait()
                pltpu.make_async_copy(
                    table_hbm.at[od_v.at[f]],
                    buf_od(par, fl),
                    sems[par],
                ).wait()

        def pool(g):
            par = g % 2
            first = g == 0
            nf = len(GROUPS[g])

            def body(b, carry):
                o = b * OUTW
                if first:
                    out_v[pl.ds(o, LANES)] = jnp.ones((LANES,), jnp.float32)
                    acc_lo = buf_ev(par, 0)[b, pl.ds(0, LANES)]
                    acc_hi = buf_od(par, 0)[b, pl.ds(0, LANES)]
                    start = 1
                else:
                    acc_lo = out_v[pl.ds(o + 1, LANES)]
                    acc_hi = out_v[pl.ds(o + 1 + LANES, LANES)]
                    start = 0
                for fl in range(start, nf):
                    acc_lo = acc_lo + buf_ev(par, fl)[b, pl.ds(0, LANES)]
                    acc_hi = acc_hi + buf_od(par, fl)[b, pl.ds(0, LANES)]
                out_v[pl.ds(o + 1, LANES)] = acc_lo
                out_v[pl.ds(o + 1 + LANES, LANES)] = acc_hi
                return carry

            lax.fori_loop(0, BPW, body, 0)

        fire(0)
        for g in range(1, len(GROUPS)):
            fire(g)
            drain(g - 1)
            pool(g - 1)
        last = len(GROUPS) - 1
        drain(last)
        pool(last)

        pltpu.sync_copy(
            out_v, out_hbm.at[pl.ds(wid * (BPW * OUTW), BPW * OUTW)]
        )

    return sc_kernel


_SC_KERNEL = _build()

TCB = 8192
WIDE = 128


def _tc_linearize_body(x0, x1, x2, x3, o_ref):
    eye = jnp.eye(D, dtype=jnp.float32)
    ts = [
        jax.lax.dot_general(
            x[...], eye, (((0,), (0,)), ((), ())),
            preferred_element_type=jnp.float32,
        )
        for x in (x0, x1, x2, x3)
    ]
    o_ref[...] = jnp.concatenate(ts, axis=1)


_TC_LINEARIZE = pl.pallas_call(
    _tc_linearize_body,
    grid=(QROWS // TCB,),
    in_specs=[
        pl.BlockSpec((D, TCB), lambda i, q=q: (0, q * (QS // TCB) + i))
        for q in range(4)
    ],
    out_specs=pl.BlockSpec((TCB, WIDE), lambda i: (i, 0)),
    out_shape=jax.ShapeDtypeStruct((QROWS, WIDE), jnp.float32),
)


@jax.jit
def kernel(user_feature_ids, embedding_weight):
    idx_t = jnp.asarray(user_feature_ids, jnp.int32).T
    table_lin = _TC_LINEARIZE(*([embedding_weight.T] * 4))
    table2 = table_lin.reshape(VR, HALF)
    flat = _SC_KERNEL(idx_t, table2)
    return flat.reshape(B, OUTW)
